# baseline (speedup 1.0000x reference)
import functools

import jax
import jax.numpy as jnp
from jax import lax
from jax.experimental import pallas as pl
from jax.experimental.pallas import tpu as pltpu
from jax.experimental.pallas import tpu_sc as plsc

_TARGET = 1536
_B, _C, _E = 8, 256, 2048
_LANES = 16
_TPB = 4
_CPT = _C // _TPB
_CCH = 16
_NCH = _CPT // _CCH


def _sort_body(len_ref, fe_ref, ord_ref, kb_ref):
    lane = lax.broadcasted_iota(jnp.int32, (1, _E), 1)
    rows = []
    for b in range(_B):
        length = len_ref[b]
        kb_ref[b] = jnp.full((1, _LANES), length - _TARGET, jnp.int32)
        fe = fe_ref[b]
        sc = jnp.sum(fe * fe, axis=0, keepdims=True)
        rows.append(jnp.where(lane < length, sc, jnp.float32(jnp.inf)))
    keys = jnp.concatenate(rows, axis=0)
    ids = jnp.broadcast_to(lane, (_B, _E))
    for p in range(11):
        for s in range(p, -1, -1):
            d = 1 << s
            ilow = (lane & d) == 0
            kp = jnp.roll(keys, -d, axis=1)
            km = jnp.roll(keys, d, axis=1)
            ip = jnp.roll(ids, -d, axis=1)
            im = jnp.roll(ids, d, axis=1)
            pk = jnp.where(ilow, kp, km)
            pid = jnp.where(ilow, ip, im)
            plt = (pk < keys) | ((pk == keys) & (pid < ids))
            dir_asc = ((lane >> (p + 1)) & 1) == 0
            take = plt ^ ilow ^ dir_asc
            keys = jnp.where(take, pk, keys)
            ids = jnp.where(take, pid, ids)
    ord_ref[...] = ids.reshape(_B, 1, _E)


def _tc_sort(fe, lengths):
    return pl.pallas_call(
        _sort_body,
        in_specs=[
            pl.BlockSpec(memory_space=pltpu.SMEM),
            pl.BlockSpec((_B, _C, _E), lambda: (0, 0, 0)),
        ],
        out_specs=[
            pl.BlockSpec((_B, 1, _E), lambda: (0, 0, 0)),
            pl.BlockSpec((_B, 1, _LANES), lambda: (0, 0, 0)),
        ],
        out_shape=[
            jax.ShapeDtypeStruct((_B, 1, _E), jnp.int32),
            jax.ShapeDtypeStruct((_B, 1, _LANES), jnp.int32),
        ],
    )(lengths, fe)


def _sc_apply_body(fe_hbm, ord_hbm, kb_hbm, out_hbm,
                   lenv, ord_v, idx1_v, idx2_v, wt_v,
                   in0, in1, ot0, ot1,
                   sin0, sin1, sout0, sout1):
    cid = lax.axis_index("c")
    sid = lax.axis_index("s")
    wid = cid * 16 + sid
    b = wid // _TPB
    cbase = (wid % _TPB) * _CPT

    iota = lax.broadcasted_iota(jnp.int32, (_LANES,), 0)

    ins = (in0, in1)
    outs = (ot0, ot1)
    isems = (sin0, sin1)
    osems = (sout0, sout1)

    pltpu.make_async_copy(
        fe_hbm.at[b, pl.ds(cbase, _CCH)], ins[0], isems[0]).start()

    pltpu.sync_copy(kb_hbm.at[b, 0], lenv)
    kvec = lenv[...]

    pltpu.sync_copy(ord_hbm.at[b, 0], ord_v)

    @plsc.parallel_loop(0, _TARGET // _LANES, unroll=4)
    def idx_body(j):
        tv = iota + j * _LANES
        idx1_v[pl.ds(j * _LANES, _LANES)] = plsc.load_gather(ord_v, [tv + kvec])
        idx2_v[pl.ds(j * _LANES, _LANES)] = ord_v[pl.ds(j * _LANES, _LANES)]
        wt_v[pl.ds(j * _LANES, _LANES)] = jnp.where(
            tv < kvec, jnp.float32(1.0), jnp.float32(0.0))

    def in_copy(ch):
        return pltpu.make_async_copy(
            fe_hbm.at[b, pl.ds(cbase + ch * _CCH, _CCH)],
            ins[ch % 2], isems[ch % 2])

    def out_copy(ch):
        return pltpu.make_async_copy(
            outs[ch % 2],
            out_hbm.at[b, pl.ds(cbase + ch * _CCH, _CCH)],
            osems[ch % 2])

    rfulls = [jnp.full((_LANES,), r, jnp.int32) for r in range(_CCH)]
    _JK = 512 // _LANES

    for ch in range(_NCH):
        if ch + 1 < _NCH:
            in_copy(ch + 1).start()
        in_copy(ch).wait()
        if ch >= 2:
            out_copy(ch - 2).wait()
        ibuf = ins[ch % 2]
        obuf = outs[ch % 2]

        @plsc.parallel_loop(0, _JK, unroll=4)
        def pair_body(j):
            sl = pl.ds(j * _LANES, _LANES)
            i1 = idx1_v[sl]
            i2 = idx2_v[sl]
            w = wt_v[sl]
            for r in range(_CCH):
                v1 = plsc.load_gather(ibuf, [rfulls[r], i1])
                v2 = plsc.load_gather(ibuf, [rfulls[r], i2])
                obuf[r, sl] = v1 + w * v2

        @plsc.parallel_loop(_JK, _TARGET // _LANES, unroll=4)
        def solo_body(j):
            sl = pl.ds(j * _LANES, _LANES)
            i1 = idx1_v[sl]
            for r in range(_CCH):
                obuf[r, sl] = plsc.load_gather(ibuf, [rfulls[r], i1])
        out_copy(ch).start()

    for ch in range(max(0, _NCH - 2), _NCH):
        out_copy(ch).wait()


_sc_apply = functools.partial(
    pl.kernel,
    out_type=jax.ShapeDtypeStruct((_B, _C, _TARGET), jnp.float32),
    mesh=plsc.VectorSubcoreMesh(core_axis_name="c", subcore_axis_name="s",
                                num_cores=2, num_subcores=16),
    compiler_params=pltpu.CompilerParams(needs_layout_passes=False),
    scratch_types=[
        pltpu.VMEM((_LANES,), jnp.int32),
        pltpu.VMEM((_E,), jnp.int32),
        pltpu.VMEM((_TARGET,), jnp.int32),
        pltpu.VMEM((_TARGET,), jnp.int32),
        pltpu.VMEM((_TARGET,), jnp.float32),
        pltpu.VMEM((_CCH, _E), jnp.float32),
        pltpu.VMEM((_CCH, _E), jnp.float32),
        pltpu.VMEM((_CCH, _TARGET), jnp.float32),
        pltpu.VMEM((_CCH, _TARGET), jnp.float32),
        pltpu.SemaphoreType.DMA,
        pltpu.SemaphoreType.DMA,
        pltpu.SemaphoreType.DMA,
        pltpu.SemaphoreType.DMA,
    ],
)(_sc_apply_body)


def kernel(fe, lengths):
    order, kb = _tc_sort(fe, lengths)
    return _sc_apply(fe, order, kb)


# speedup vs baseline: 1.0976x; 1.0976x over previous
import functools

import jax
import jax.numpy as jnp
from jax import lax
from jax.experimental import pallas as pl
from jax.experimental.pallas import tpu as pltpu
from jax.experimental.pallas import tpu_sc as plsc

_TARGET = 1536
_B, _C, _E = 8, 256, 2048
_LANES = 16
_TPB = 4
_CPT = _C // _TPB
_CCH = 16
_NCH = _CPT // _CCH


def _sort_body(len_ref, fe_ref, ord_ref, kb_ref):
    lane = lax.broadcasted_iota(jnp.int32, (1, _E), 1)
    rows = []
    for b in range(_B):
        length = len_ref[b]
        kb_ref[b] = jnp.full((1, _LANES), length - _TARGET, jnp.int32)
        fe = fe_ref[b]
        sc = jnp.sum(fe * fe, axis=0, keepdims=True)
        rows.append(jnp.where(lane < length, sc, jnp.float32(jnp.inf)))
    keys = jnp.concatenate(rows, axis=0)
    ids = jnp.broadcast_to(lane, (_B, _E))
    h_w = _E // 2
    kh = [keys[:, :h_w], keys[:, h_w:]]
    ih = [ids[:, :h_w], ids[:, h_w:]]
    lh = [lax.broadcasted_iota(jnp.int32, (1, h_w), 1),
          lax.broadcasted_iota(jnp.int32, (1, h_w), 1) + h_w]
    for p in range(11):
        for s in range(p, -1, -1):
            d = 1 << s
            if d == h_w:
                m = (kh[1] < kh[0]) | ((kh[1] == kh[0]) & (ih[1] < ih[0]))
                kh = [jnp.where(m, kh[1], kh[0]), jnp.where(m, kh[0], kh[1])]
                ih = [jnp.where(m, ih[1], ih[0]), jnp.where(m, ih[0], ih[1])]
            else:
                for h in (0, 1):
                    L = lh[h]
                    ilow = (L & d) == 0
                    kp = jnp.roll(kh[h], -d, axis=1)
                    km = jnp.roll(kh[h], d, axis=1)
                    ip = jnp.roll(ih[h], -d, axis=1)
                    im = jnp.roll(ih[h], d, axis=1)
                    pk = jnp.where(ilow, kp, km)
                    pid = jnp.where(ilow, ip, im)
                    plt = (pk < kh[h]) | ((pk == kh[h]) & (pid < ih[h]))
                    dir_asc = ((L >> (p + 1)) & 1) == 0
                    take = plt ^ ilow ^ dir_asc
                    kh[h] = jnp.where(take, pk, kh[h])
                    ih[h] = jnp.where(take, pid, ih[h])
    ord_ref[...] = jnp.concatenate(ih, axis=1).reshape(_B, 1, _E)


def _tc_sort(fe, lengths):
    return pl.pallas_call(
        _sort_body,
        in_specs=[
            pl.BlockSpec(memory_space=pltpu.SMEM),
            pl.BlockSpec((_B, _C, _E), lambda: (0, 0, 0)),
        ],
        out_specs=[
            pl.BlockSpec((_B, 1, _E), lambda: (0, 0, 0)),
            pl.BlockSpec((_B, 1, _LANES), lambda: (0, 0, 0)),
        ],
        out_shape=[
            jax.ShapeDtypeStruct((_B, 1, _E), jnp.int32),
            jax.ShapeDtypeStruct((_B, 1, _LANES), jnp.int32),
        ],
    )(lengths, fe)


def _sc_apply_body(fe_hbm, ord_hbm, kb_hbm, out_hbm,
                   lenv, ord_v, idx1_v, idx2_v, wt_v,
                   in0, in1, ot0, ot1,
                   sin0, sin1, sout0, sout1):
    cid = lax.axis_index("c")
    sid = lax.axis_index("s")
    wid = cid * 16 + sid
    b = wid // _TPB
    cbase = (wid % _TPB) * _CPT

    iota = lax.broadcasted_iota(jnp.int32, (_LANES,), 0)

    ins = (in0, in1)
    outs = (ot0, ot1)
    isems = (sin0, sin1)
    osems = (sout0, sout1)

    pltpu.make_async_copy(
        fe_hbm.at[b, pl.ds(cbase, _CCH)], ins[0], isems[0]).start()

    pltpu.sync_copy(kb_hbm.at[b, 0], lenv)
    kvec = lenv[...]

    pltpu.sync_copy(ord_hbm.at[b, 0], ord_v)

    @plsc.parallel_loop(0, _TARGET // _LANES, unroll=4)
    def idx_body(j):
        tv = iota + j * _LANES
        idx1_v[pl.ds(j * _LANES, _LANES)] = plsc.load_gather(ord_v, [tv + kvec])
        idx2_v[pl.ds(j * _LANES, _LANES)] = ord_v[pl.ds(j * _LANES, _LANES)]
        wt_v[pl.ds(j * _LANES, _LANES)] = jnp.where(
            tv < kvec, jnp.float32(1.0), jnp.float32(0.0))

    def in_copy(ch):
        return pltpu.make_async_copy(
            fe_hbm.at[b, pl.ds(cbase + ch * _CCH, _CCH)],
            ins[ch % 2], isems[ch % 2])

    def out_copy(ch):
        return pltpu.make_async_copy(
            outs[ch % 2],
            out_hbm.at[b, pl.ds(cbase + ch * _CCH, _CCH)],
            osems[ch % 2])

    rfulls = [jnp.full((_LANES,), r, jnp.int32) for r in range(_CCH)]
    _JK = 512 // _LANES

    for ch in range(_NCH):
        if ch + 1 < _NCH:
            in_copy(ch + 1).start()
        in_copy(ch).wait()
        if ch >= 2:
            out_copy(ch - 2).wait()
        ibuf = ins[ch % 2]
        obuf = outs[ch % 2]

        @plsc.parallel_loop(0, _JK, unroll=2)
        def pair_body(j):
            sl = pl.ds(j * _LANES, _LANES)
            i1 = idx1_v[sl]
            i2 = idx2_v[sl]
            w = wt_v[sl]
            for r in range(_CCH):
                v1 = plsc.load_gather(ibuf, [rfulls[r], i1])
                v2 = plsc.load_gather(ibuf, [rfulls[r], i2])
                obuf[r, sl] = v1 + w * v2

        @plsc.parallel_loop(_JK, _TARGET // _LANES, unroll=2)
        def solo_body(j):
            sl = pl.ds(j * _LANES, _LANES)
            i1 = idx1_v[sl]
            for r in range(_CCH):
                obuf[r, sl] = plsc.load_gather(ibuf, [rfulls[r], i1])
        out_copy(ch).start()

    for ch in range(max(0, _NCH - 2), _NCH):
        out_copy(ch).wait()


_sc_apply = functools.partial(
    pl.kernel,
    out_type=jax.ShapeDtypeStruct((_B, _C, _TARGET), jnp.float32),
    mesh=plsc.VectorSubcoreMesh(core_axis_name="c", subcore_axis_name="s",
                                num_cores=2, num_subcores=16),
    compiler_params=pltpu.CompilerParams(needs_layout_passes=False),
    scratch_types=[
        pltpu.VMEM((_LANES,), jnp.int32),
        pltpu.VMEM((_E,), jnp.int32),
        pltpu.VMEM((_TARGET,), jnp.int32),
        pltpu.VMEM((_TARGET,), jnp.int32),
        pltpu.VMEM((_TARGET,), jnp.float32),
        pltpu.VMEM((_CCH, _E), jnp.float32),
        pltpu.VMEM((_CCH, _E), jnp.float32),
        pltpu.VMEM((_CCH, _TARGET), jnp.float32),
        pltpu.VMEM((_CCH, _TARGET), jnp.float32),
        pltpu.SemaphoreType.DMA,
        pltpu.SemaphoreType.DMA,
        pltpu.SemaphoreType.DMA,
        pltpu.SemaphoreType.DMA,
    ],
)(_sc_apply_body)


def kernel(fe, lengths):
    order, kb = _tc_sort(fe, lengths)
    return _sc_apply(fe, order, kb)
